# baseline (speedup 1.0000x reference)
import functools
import math

import jax
import jax.numpy as jnp
from jax import lax
from jax.experimental import pallas as pl
from jax.experimental.pallas import tpu as pltpu
from jax.experimental.pallas import tpu_sc as plsc

EMBED_DIM = 64
SEQ_LEN = 200
NUM_CORES = 2
NUM_SUBCORES = 16
NUM_WORKERS = NUM_CORES * NUM_SUBCORES
LANES = 16
CHUNK = 400


def _make_sinusoidal_pe(seq_len, dim):
    position = jnp.arange(0, seq_len, dtype=jnp.float32)[:, None]
    div_term = jnp.exp(
        jnp.arange(0, dim, 2, dtype=jnp.float32) * -(math.log(10000.0) / dim)
    )
    pe = jnp.zeros((seq_len, dim), dtype=jnp.float32)
    pe = pe.at[:, 0::2].set(jnp.sin(position * div_term))
    pe = pe.at[:, 1::2].set(jnp.cos(position * div_term))
    return pe


def _sc_embed(ids_flat, table, pe_chunk, *, dim, chunk, num_cores, num_subcores,
              interpret=False):
    num_workers = num_cores * num_subcores
    b = ids_flat.shape[0]
    b_per_w = b // num_workers
    n_chunks = b_per_w // chunk
    mesh = plsc.VectorSubcoreMesh(
        core_axis_name="c", subcore_axis_name="s",
        num_cores=num_cores, num_subcores=num_subcores,
    )

    @functools.partial(
        pl.kernel,
        out_type=jax.ShapeDtypeStruct((b, dim), jnp.float32),
        mesh=mesh,
        scratch_types=[
            pltpu.VMEM((chunk,), jnp.int32),
            pltpu.VMEM((chunk, dim), jnp.float32),
            pltpu.VMEM((chunk, dim), jnp.float32),
            pltpu.SemaphoreType.DMA,
        ],
        compiler_params=pltpu.CompilerParams(use_tc_tiling_on_sc=False),
        interpret=interpret,
    )
    def run(ids_hbm, table_hbm, pe_hbm, out_hbm, idx_v, rows_v, pe_v, sem):
        wid = lax.axis_index("s") * num_cores + lax.axis_index("c")
        base = wid * b_per_w
        pltpu.sync_copy(pe_hbm, pe_v)

        @pl.loop(0, n_chunks)
        def _chunk_loop(g):
            row0 = base + g * chunk
            pltpu.sync_copy(ids_hbm.at[pl.ds(row0, chunk)], idx_v)
            pltpu.async_copy(table_hbm.at[idx_v], rows_v, sem).wait()

            @pl.loop(0, chunk)
            def _row_loop(r):
                for c in range(dim // LANES):
                    plsc.addupdate(
                        rows_v.at[r, pl.ds(c * LANES, LANES)],
                        pe_v[r, pl.ds(c * LANES, LANES)],
                    )

            pltpu.sync_copy(rows_v, out_hbm.at[pl.ds(row0, chunk)])

    return run(ids_flat, table, pe_chunk)


def kernel(input_ids, token_embedding):
    batch, seq_len = input_ids.shape
    dim = token_embedding.shape[1]
    ids_flat = input_ids.reshape(-1).astype(jnp.int32)
    pe = _make_sinusoidal_pe(seq_len, dim)
    reps = CHUNK // seq_len
    pe_chunk = jnp.concatenate([pe] * reps, axis=0)
    out = _sc_embed(
        ids_flat, token_embedding, pe_chunk,
        dim=dim, chunk=CHUNK, num_cores=NUM_CORES, num_subcores=NUM_SUBCORES,
    )
    return out.reshape(batch, seq_len, dim)
